# baseline (speedup 1.0000x reference)
import jax
import jax.numpy as jnp
from jax import lax
from jax.experimental import pallas as pl
from jax.experimental.pallas import tpu as pltpu
from jax.experimental.pallas import tpu_sc as plsc

_N = 50000
_NPAD = 50176
_E = 1600000
_B = 512
_F = 8
_W = 16
_EW = 125
_NROWS = _E // _EW
_RPW = _NROWS // 32
_ICH = 40
_NCH = _RPW // _ICH
_TS = _NPAD // 16
_NB = 512
_UNR = 2
_NBLK = _NPAD // (_NB * _UNR)


def _sc_agg_body(xpad_hbm, src_hbm, dst_hbm, zrows_hbm, out_hbm,
                 sidx, didx, r0, r1, r2, r3, shared,
                 g0, g1, g2, g3, s0, s1, s2, s3):
    rows = [r0, r1, r2, r3]
    gsem = [g0, g1, g2, g3]
    ssem = [s0, s1, s2, s3]
    c = lax.axis_index("c")
    s = lax.axis_index("s")
    wid = c * 16 + s
    pltpu.sync_copy(zrows_hbm, shared.at[pl.ds(s * _TS, _TS)])
    plsc.subcore_barrier()
    row0 = wid * _RPW

    nbuf = len(rows)

    def outer(i, carry):
        base = row0 + i * _ICH
        pltpu.sync_copy(src_hbm.at[pl.ds(base, _ICH)], sidx)
        pltpu.sync_copy(dst_hbm.at[pl.ds(base, _ICH)], didx)

        for b in range(nbuf):
            pltpu.async_copy(xpad_hbm.at[sidx.at[b]], rows[b], gsem[b])

        def inner(k, carry2):
            j0 = nbuf * k
            for b in range(nbuf):
                pltpu.make_async_copy(xpad_hbm.at[sidx.at[j0 + b]], rows[b],
                                      gsem[b]).wait()
                pltpu.async_copy(rows[b], shared.at[didx.at[j0 + b]],
                                 ssem[b], add=True)
            for b in range(nbuf):
                jn = j0 + nbuf + b

                @pl.when(jn < _ICH)
                def _nxt():
                    pltpu.make_async_copy(rows[b], shared.at[didx.at[j0 + b]],
                                          ssem[b]).wait()
                    pltpu.async_copy(xpad_hbm.at[sidx.at[jn]], rows[b],
                                     gsem[b])

            return carry2

        lax.fori_loop(0, _ICH // nbuf, inner, 0)
        for b in range(nbuf):
            pltpu.make_async_copy(rows[b], shared.at[didx.at[b]],
                                  ssem[b]).wait()
        return carry

    lax.fori_loop(0, _NCH, outer, 0)
    plsc.subcore_barrier()
    pltpu.sync_copy(shared.at[pl.ds(s * _TS, _TS)],
                    out_hbm.at[c, pl.ds(s * _TS, _TS)])


def _build_sc_agg(interpret=False):
    return pl.kernel(
        _sc_agg_body,
        out_type=jax.ShapeDtypeStruct((2, _NPAD, _W), jnp.float32),
        mesh=plsc.VectorSubcoreMesh(core_axis_name="c", subcore_axis_name="s"),
        scratch_types=[
            pltpu.VMEM((_ICH, _EW), jnp.int32),
            pltpu.VMEM((_ICH, _EW), jnp.int32),
            pltpu.VMEM((_EW, _W), jnp.float32),
            pltpu.VMEM((_EW, _W), jnp.float32),
            pltpu.VMEM((_EW, _W), jnp.float32),
            pltpu.VMEM((_EW, _W), jnp.float32),
            pltpu.VMEM_SHARED((_NPAD, _W), jnp.float32),
            pltpu.SemaphoreType.DMA,
            pltpu.SemaphoreType.DMA,
            pltpu.SemaphoreType.DMA,
            pltpu.SemaphoreType.DMA,
            pltpu.SemaphoreType.DMA,
            pltpu.SemaphoreType.DMA,
            pltpu.SemaphoreType.DMA,
            pltpu.SemaphoreType.DMA,
        ],
        compiler_params=pltpu.CompilerParams(use_tc_tiling_on_sc=False),
        interpret=interpret,
    )


def _onehot_t(batch_ref, i):
    brow = batch_ref[:, pl.ds(i * _NB, _NB)]
    io = lax.broadcasted_iota(jnp.int32, (_B, _NB), 0)
    return jnp.where(io == brow, 1.0, 0.0)


def _seg_dot(data, oht):
    return lax.dot_general(data, oht, (((1,), (1,)), ((), ())),
                           preferred_element_type=jnp.float32)


def _to_nodes(pergraph, oht):
    return lax.dot_general(pergraph, oht, (((1,), (0,)), ((), ())),
                           preferred_element_type=jnp.float32)


def _lstm_cell(wih, whh, bih, bhh, qs, h, cst):
    gates = (jnp.dot(wih, qs, preferred_element_type=jnp.float32) + bih
             + jnp.dot(whh, h, preferred_element_type=jnp.float32) + bhh)
    ig = jax.nn.sigmoid(gates[0:8, :])
    fg = jax.nn.sigmoid(gates[8:16, :])
    gg = jnp.tanh(gates[16:24, :])
    og = jax.nn.sigmoid(gates[24:32, :])
    cst = fg * cst + ig * gg
    return og * jnp.tanh(cst), cst


def _layer_common(part, xprev, batch_ref, wlt_r, bl_r, wrt_r, wih_r, whh_r,
                  bih_r, bhh_r, xs_ref, gn=None):
    wlt = wlt_r[...]
    bl = bl_r[...]
    wrt = wrt_r[...]
    wih = wih_r[...]
    whh = whh_r[...]
    bih = bih_r[...]
    bhh = bhh_r[...]
    do_gn = gn is not None
    if do_gn:
        gnw, gnb, gnms, xnext_ref = gn
    ones_row = jnp.ones((1, _NB), jnp.float32)
    zc = jnp.zeros((1, _B), jnp.float32)
    zf = jnp.zeros((_F, _B), jnp.float32)

    h0, c0 = _lstm_cell(wih, whh, bih, bhh,
                        jnp.zeros((2 * _F, 1), jnp.float32),
                        jnp.zeros((_F, 1), jnp.float32),
                        jnp.zeros((_F, 1), jnp.float32))

    def p1(i, carry):
        cnt, msum, asum, rnum = carry
        for u in range(_UNR):
            j = i * _UNR + u
            sl = pl.ds(j * _NB, _NB)
            agg = part[0, :, sl] + part[1, :, sl]
            deg = agg[_F:_F + 1, :]
            mean = agg[0:_F, :] / jnp.maximum(deg, 1.0)
            xp = xprev[0:_F, sl]
            hh = (jnp.dot(wlt, mean, preferred_element_type=jnp.float32)
                  + jnp.dot(wrt, xp, preferred_element_type=jnp.float32)
                  + bl)
            xsb = jnp.tanh(hh)
            xs_ref[:, sl] = xsb
            oht = _onehot_t(batch_ref, j)
            a = jnp.exp(jnp.sum(xsb * h0, axis=0, keepdims=True))
            asum = asum + _seg_dot(a, oht)
            rnum = rnum + _seg_dot(a * xsb, oht)
            if do_gn:
                cnt = cnt + _seg_dot(ones_row, oht)
                msum = msum + _seg_dot(xsb, oht)
        return cnt, msum, asum, rnum

    cnt, msum, asum, rnum = lax.fori_loop(0, _NBLK, p1, (zc, zf, zc, zf))
    cntc = jnp.maximum(cnt, 1.0)
    mean = msum / cntc
    h0b = h0 + jnp.zeros((_F, _B), jnp.float32)
    qs = jnp.concatenate([h0b, rnum / (asum + 1e-16)], axis=0)

    h1, c1 = _lstm_cell(wih, whh, bih, bhh, qs, h0b, c0)

    def p2(i, carry):
        asum, rnum, vsum = carry
        for u in range(_UNR):
            j = i * _UNR + u
            sl = pl.ds(j * _NB, _NB)
            xsb = xs_ref[:, sl]
            oht = _onehot_t(batch_ref, j)
            qb = _to_nodes(h1, oht)
            a = jnp.exp(jnp.sum(xsb * qb, axis=0, keepdims=True))
            asum = asum + _seg_dot(a, oht)
            rnum = rnum + _seg_dot(a * xsb, oht)
            if do_gn:
                ob = xsb - _to_nodes(mean, oht) * gnms
                vsum = vsum + _seg_dot(ob * ob, oht)
        return asum, rnum, vsum

    asum, rnum, vsum = lax.fori_loop(0, _NBLK, p2, (zc, zf, zf))
    qs = jnp.concatenate([h1, rnum / (asum + 1e-16)], axis=0)

    h2, c2 = _lstm_cell(wih, whh, bih, bhh, qs, h1, c1)
    if do_gn:
        rstd = lax.rsqrt(vsum / cntc + 1e-5)
        pad1 = jnp.ones((1, _NB), jnp.float32)
        pad0 = jnp.zeros((_W - _F - 1, _NB), jnp.float32)

    def p3(i, carry):
        asum, rnum = carry
        for u in range(_UNR):
            j = i * _UNR + u
            sl = pl.ds(j * _NB, _NB)
            xsb = xs_ref[:, sl]
            oht = _onehot_t(batch_ref, j)
            qb = _to_nodes(h2, oht)
            a = jnp.exp(jnp.sum(xsb * qb, axis=0, keepdims=True))
            asum = asum + _seg_dot(a, oht)
            rnum = rnum + _seg_dot(a * xsb, oht)
            if do_gn:
                ob = xsb - _to_nodes(mean, oht) * gnms
                y = gnw * ob * _to_nodes(rstd, oht) + gnb
                xnext_ref[:, sl] = jnp.concatenate([y, pad1, pad0], axis=0)
        return asum, rnum

    asum, rnum = lax.fori_loop(0, _NBLK, p3, (zc, zf))
    return jnp.concatenate([h2, rnum / (asum + 1e-16)], axis=0)


def _mid_layer_body(part, xprev, batch_ref, wlt, bl, wrt, wih, whh, bih, bhh,
                    gnw_r, gnb_r, gnms_r, qstar_ref, xnext_ref, xs_ref):
    qstar_ref[...] = _layer_common(
        part, xprev, batch_ref, wlt, bl, wrt, wih, whh, bih, bhh, xs_ref,
        gn=(gnw_r[...], gnb_r[...], gnms_r[...], xnext_ref))


def _last_layer_body(part, xprev, batch_ref, wlt, bl, wrt, wih, whh, bih,
                     bhh, q1, q2, outw, outb, final_ref, xs_ref):
    qs = _layer_common(
        part, xprev, batch_ref, wlt, bl, wrt, wih, whh, bih, bhh, xs_ref)
    cat = jnp.concatenate([q1[...], q2[...], qs], axis=0)
    s = jnp.sum(cat * outw[...], axis=0, keepdims=True)
    final_ref[...] = jnp.tanh(s + outb[...])


def _build_mid(interpret=False):
    return pl.pallas_call(
        _mid_layer_body,
        out_shape=[jax.ShapeDtypeStruct((2 * _F, _B), jnp.float32),
                   jax.ShapeDtypeStruct((_W, _NPAD), jnp.float32)],
        scratch_shapes=[pltpu.VMEM((_F, _NPAD), jnp.float32)],
        interpret=interpret,
    )


def _build_last(interpret=False):
    return pl.pallas_call(
        _last_layer_body,
        out_shape=jax.ShapeDtypeStruct((1, _B), jnp.float32),
        scratch_shapes=[pltpu.VMEM((_F, _NPAD), jnp.float32)],
        interpret=interpret,
    )


_PB = 1024


def _prep_body(x_ref, w_ref, b_ref, out_ref):
    xb = x_ref[...]
    h = jnp.tanh(w_ref[...] * xb + b_ref[...])
    out_ref[...] = jnp.concatenate(
        [h, jnp.ones((1, _PB), jnp.float32),
         jnp.zeros((_W - _F - 1, _PB), jnp.float32)], axis=0)


def _build_prep(interpret=False):
    return pl.pallas_call(
        _prep_body,
        grid=(_NPAD // _PB,),
        in_specs=[pl.BlockSpec((1, _PB), lambda i: (0, i)),
                  pl.BlockSpec((_F, 1), lambda i: (0, 0)),
                  pl.BlockSpec((_F, 1), lambda i: (0, 0))],
        out_specs=pl.BlockSpec((_W, _PB), lambda i: (0, i)),
        out_shape=jax.ShapeDtypeStruct((_W, _NPAD), jnp.float32),
        interpret=interpret,
    )


_mid = _build_mid()
_last = _build_last()
_prep = _build_prep()


def kernel(x, edge_index, batch, in_w, in_b, c1_wl, c1_bl, c1_wr, c2_wl,
           c2_bl, c2_wr, c3_wl, c3_bl, c3_wr, gn2_w, gn2_b, gn2_ms, gn3_w,
           gn3_b, gn3_ms, s1_wih, s1_whh, s1_bih, s1_bhh, s2_wih, s2_whh,
           s2_bih, s2_bhh, s3_wih, s3_whh, s3_bih, s3_bhh, out_w, out_b):
    f32 = jnp.float32
    _sc_agg = _build_sc_agg()
    src2d = edge_index[0].reshape(_NROWS, _EW)
    dst2d = edge_index[1].reshape(_NROWS, _EW)
    batcht = jnp.pad(batch.reshape(1, _N), ((0, 0), (0, _NPAD - _N)),
                     constant_values=_B)
    zrows = jnp.zeros((_TS, _W), f32)
    xt = jnp.pad(x.reshape(1, _N), ((0, 0), (0, _NPAD - _N)))

    def col(v):
        return v.reshape(-1, 1)

    xpadt = _prep(xt, col(in_w.reshape(_F)), col(in_b))

    out = None
    q1 = q2 = None
    layer = [
        (c1_wl, c1_bl, c1_wr, s1_wih, s1_whh, s1_bih, s1_bhh,
         gn2_w, gn2_b, gn2_ms),
        (c2_wl, c2_bl, c2_wr, s2_wih, s2_whh, s2_bih, s2_bhh,
         gn3_w, gn3_b, gn3_ms),
        (c3_wl, c3_bl, c3_wr, s3_wih, s3_whh, s3_bih, s3_bhh,
         None, None, None),
    ]
    for li, (wl, bl, wr, wih, whh, bih, bhh, gw, gb, gms) in enumerate(layer):
        xpad_nm = xpadt.T
        part = _sc_agg(xpad_nm, src2d, dst2d, zrows)
        partt = jnp.transpose(part, (0, 2, 1))
        if li < 2:
            qs, xpadt = _mid(partt, xpadt, batcht, wl.T, col(bl), wr.T,
                             wih, whh, col(bih), col(bhh),
                             col(gw), col(gb), col(gms))
            if li == 0:
                q1 = qs
            else:
                q2 = qs
        else:
            out = _last(partt, xpadt, batcht, wl.T, col(bl), wr.T,
                        wih, whh, col(bih), col(bhh),
                        q1, q2, col(out_w.reshape(-1)), out_b.reshape(1, 1))
    return out.reshape(_B, 1)


# speedup vs baseline: 1.0529x; 1.0529x over previous
import jax
import jax.numpy as jnp
from jax import lax
from jax.experimental import pallas as pl
from jax.experimental.pallas import tpu as pltpu
from jax.experimental.pallas import tpu_sc as plsc

_N = 50000
_NPAD = 51200
_E = 1600000
_B = 512
_F = 8
_W = 16
_EW = 125
_NROWS = _E // _EW
_RPW = _NROWS // 32
_ICH = 40
_NCH = _RPW // _ICH
_TS = _NPAD // 16
_NB = 1024
_UNR = 2
_NBLK = _NPAD // (_NB * _UNR)


def _sc_agg_body(xpad_hbm, src_hbm, dst_hbm, zrows_hbm, out_hbm,
                 sidx, didx, r0, r1, r2, r3, shared,
                 g0, g1, g2, g3, s0, s1, s2, s3):
    rows = [r0, r1, r2, r3]
    gsem = [g0, g1, g2, g3]
    ssem = [s0, s1, s2, s3]
    c = lax.axis_index("c")
    s = lax.axis_index("s")
    wid = c * 16 + s
    pltpu.sync_copy(zrows_hbm, shared.at[pl.ds(s * _TS, _TS)])
    plsc.subcore_barrier()
    row0 = wid * _RPW

    nbuf = len(rows)

    def outer(i, carry):
        base = row0 + i * _ICH
        pltpu.sync_copy(src_hbm.at[pl.ds(base, _ICH)], sidx)
        pltpu.sync_copy(dst_hbm.at[pl.ds(base, _ICH)], didx)

        for b in range(nbuf):
            pltpu.async_copy(xpad_hbm.at[sidx.at[b]], rows[b], gsem[b])

        def inner(k, carry2):
            j0 = nbuf * k
            for b in range(nbuf):
                pltpu.make_async_copy(xpad_hbm.at[sidx.at[j0 + b]], rows[b],
                                      gsem[b]).wait()
                pltpu.async_copy(rows[b], shared.at[didx.at[j0 + b]],
                                 ssem[b], add=True)
            for b in range(nbuf):
                jn = j0 + nbuf + b

                @pl.when(jn < _ICH)
                def _nxt():
                    pltpu.make_async_copy(rows[b], shared.at[didx.at[j0 + b]],
                                          ssem[b]).wait()
                    pltpu.async_copy(xpad_hbm.at[sidx.at[jn]], rows[b],
                                     gsem[b])

            return carry2

        lax.fori_loop(0, _ICH // nbuf, inner, 0)
        for b in range(nbuf):
            pltpu.make_async_copy(rows[b], shared.at[didx.at[b]],
                                  ssem[b]).wait()
        return carry

    lax.fori_loop(0, _NCH, outer, 0)
    plsc.subcore_barrier()
    pltpu.sync_copy(shared.at[pl.ds(s * _TS, _TS)],
                    out_hbm.at[c, pl.ds(s * _TS, _TS)])


def _build_sc_agg(interpret=False):
    return pl.kernel(
        _sc_agg_body,
        out_type=jax.ShapeDtypeStruct((2, _NPAD, _W), jnp.float32),
        mesh=plsc.VectorSubcoreMesh(core_axis_name="c", subcore_axis_name="s"),
        scratch_types=[
            pltpu.VMEM((_ICH, _EW), jnp.int32),
            pltpu.VMEM((_ICH, _EW), jnp.int32),
            pltpu.VMEM((_EW, _W), jnp.float32),
            pltpu.VMEM((_EW, _W), jnp.float32),
            pltpu.VMEM((_EW, _W), jnp.float32),
            pltpu.VMEM((_EW, _W), jnp.float32),
            pltpu.VMEM_SHARED((_NPAD, _W), jnp.float32),
            pltpu.SemaphoreType.DMA,
            pltpu.SemaphoreType.DMA,
            pltpu.SemaphoreType.DMA,
            pltpu.SemaphoreType.DMA,
            pltpu.SemaphoreType.DMA,
            pltpu.SemaphoreType.DMA,
            pltpu.SemaphoreType.DMA,
            pltpu.SemaphoreType.DMA,
        ],
        compiler_params=pltpu.CompilerParams(use_tc_tiling_on_sc=False),
        interpret=interpret,
    )


def _onehot_t(batch_ref, i):
    brow = batch_ref[:, pl.ds(i * _NB, _NB)]
    io = lax.broadcasted_iota(jnp.int32, (_B, _NB), 0)
    return jnp.where(io == brow, 1.0, 0.0)


def _seg_dot(data, oht):
    return lax.dot_general(data, oht, (((1,), (1,)), ((), ())),
                           preferred_element_type=jnp.float32)


def _to_nodes(pergraph, oht):
    return lax.dot_general(pergraph, oht, (((1,), (0,)), ((), ())),
                           preferred_element_type=jnp.float32)


def _lstm_cell(wih, whh, bih, bhh, qs, h, cst):
    gates = (jnp.dot(wih, qs, preferred_element_type=jnp.float32) + bih
             + jnp.dot(whh, h, preferred_element_type=jnp.float32) + bhh)
    ig = jax.nn.sigmoid(gates[0:8, :])
    fg = jax.nn.sigmoid(gates[8:16, :])
    gg = jnp.tanh(gates[16:24, :])
    og = jax.nn.sigmoid(gates[24:32, :])
    cst = fg * cst + ig * gg
    return og * jnp.tanh(cst), cst


def _layer_common(part, xprev, batch_ref, wlt_r, bl_r, wrt_r, wih_r, whh_r,
                  bih_r, bhh_r, xs_ref, gn=None):
    wlt = wlt_r[...]
    bl = bl_r[...]
    wrt = wrt_r[...]
    wih = wih_r[...]
    whh = whh_r[...]
    bih = bih_r[...]
    bhh = bhh_r[...]
    do_gn = gn is not None
    if do_gn:
        gnw, gnb, gnms, xnext_ref = gn
    ones_row = jnp.ones((1, _NB), jnp.float32)
    zc = jnp.zeros((1, _B), jnp.float32)
    zf = jnp.zeros((_F, _B), jnp.float32)

    h0, c0 = _lstm_cell(wih, whh, bih, bhh,
                        jnp.zeros((2 * _F, 1), jnp.float32),
                        jnp.zeros((_F, 1), jnp.float32),
                        jnp.zeros((_F, 1), jnp.float32))

    def p1(i, carry):
        cnt, msum, asum, rnum = carry
        for u in range(_UNR):
            j = i * _UNR + u
            sl = pl.ds(j * _NB, _NB)
            agg = part[0, :, sl] + part[1, :, sl]
            deg = agg[_F:_F + 1, :]
            mean = agg[0:_F, :] / jnp.maximum(deg, 1.0)
            xp = xprev[0:_F, sl]
            hh = (jnp.dot(wlt, mean, preferred_element_type=jnp.float32)
                  + jnp.dot(wrt, xp, preferred_element_type=jnp.float32)
                  + bl)
            xsb = jnp.tanh(hh)
            xs_ref[:, sl] = xsb
            oht = _onehot_t(batch_ref, j)
            a = jnp.exp(jnp.sum(xsb * h0, axis=0, keepdims=True))
            asum = asum + _seg_dot(a, oht)
            rnum = rnum + _seg_dot(a * xsb, oht)
            if do_gn:
                cnt = cnt + _seg_dot(ones_row, oht)
                msum = msum + _seg_dot(xsb, oht)
        return cnt, msum, asum, rnum

    cnt, msum, asum, rnum = lax.fori_loop(0, _NBLK, p1, (zc, zf, zc, zf))
    cntc = jnp.maximum(cnt, 1.0)
    mean = msum / cntc
    h0b = h0 + jnp.zeros((_F, _B), jnp.float32)
    qs = jnp.concatenate([h0b, rnum / (asum + 1e-16)], axis=0)

    h1, c1 = _lstm_cell(wih, whh, bih, bhh, qs, h0b, c0)

    def p2(i, carry):
        asum, rnum, vsum = carry
        for u in range(_UNR):
            j = i * _UNR + u
            sl = pl.ds(j * _NB, _NB)
            xsb = xs_ref[:, sl]
            oht = _onehot_t(batch_ref, j)
            qb = _to_nodes(h1, oht)
            a = jnp.exp(jnp.sum(xsb * qb, axis=0, keepdims=True))
            asum = asum + _seg_dot(a, oht)
            rnum = rnum + _seg_dot(a * xsb, oht)
            if do_gn:
                ob = xsb - _to_nodes(mean, oht) * gnms
                vsum = vsum + _seg_dot(ob * ob, oht)
        return asum, rnum, vsum

    asum, rnum, vsum = lax.fori_loop(0, _NBLK, p2, (zc, zf, zf))
    qs = jnp.concatenate([h1, rnum / (asum + 1e-16)], axis=0)

    h2, c2 = _lstm_cell(wih, whh, bih, bhh, qs, h1, c1)
    if do_gn:
        rstd = lax.rsqrt(vsum / cntc + 1e-5)
        pad1 = jnp.ones((1, _NB), jnp.float32)
        pad0 = jnp.zeros((_W - _F - 1, _NB), jnp.float32)

    def p3(i, carry):
        asum, rnum = carry
        for u in range(_UNR):
            j = i * _UNR + u
            sl = pl.ds(j * _NB, _NB)
            xsb = xs_ref[:, sl]
            oht = _onehot_t(batch_ref, j)
            qb = _to_nodes(h2, oht)
            a = jnp.exp(jnp.sum(xsb * qb, axis=0, keepdims=True))
            asum = asum + _seg_dot(a, oht)
            rnum = rnum + _seg_dot(a * xsb, oht)
            if do_gn:
                ob = xsb - _to_nodes(mean, oht) * gnms
                y = gnw * ob * _to_nodes(rstd, oht) + gnb
                xnext_ref[:, sl] = jnp.concatenate([y, pad1, pad0], axis=0)
        return asum, rnum

    asum, rnum = lax.fori_loop(0, _NBLK, p3, (zc, zf))
    return jnp.concatenate([h2, rnum / (asum + 1e-16)], axis=0)


def _mid_layer_body(part, xprev, batch_ref, wlt, bl, wrt, wih, whh, bih, bhh,
                    gnw_r, gnb_r, gnms_r, qstar_ref, xnext_ref, xs_ref):
    qstar_ref[...] = _layer_common(
        part, xprev, batch_ref, wlt, bl, wrt, wih, whh, bih, bhh, xs_ref,
        gn=(gnw_r[...], gnb_r[...], gnms_r[...], xnext_ref))


def _last_layer_body(part, xprev, batch_ref, wlt, bl, wrt, wih, whh, bih,
                     bhh, q1, q2, outw, outb, final_ref, xs_ref):
    qs = _layer_common(
        part, xprev, batch_ref, wlt, bl, wrt, wih, whh, bih, bhh, xs_ref)
    cat = jnp.concatenate([q1[...], q2[...], qs], axis=0)
    s = jnp.sum(cat * outw[...], axis=0, keepdims=True)
    final_ref[...] = jnp.tanh(s + outb[...])


def _build_mid(interpret=False):
    return pl.pallas_call(
        _mid_layer_body,
        out_shape=[jax.ShapeDtypeStruct((2 * _F, _B), jnp.float32),
                   jax.ShapeDtypeStruct((_W, _NPAD), jnp.float32)],
        scratch_shapes=[pltpu.VMEM((_F, _NPAD), jnp.float32)],
        interpret=interpret,
    )


def _build_last(interpret=False):
    return pl.pallas_call(
        _last_layer_body,
        out_shape=jax.ShapeDtypeStruct((1, _B), jnp.float32),
        scratch_shapes=[pltpu.VMEM((_F, _NPAD), jnp.float32)],
        interpret=interpret,
    )


_PB = 1024


def _prep_body(x_ref, w_ref, b_ref, out_ref):
    xb = x_ref[...]
    h = jnp.tanh(w_ref[...] * xb + b_ref[...])
    out_ref[...] = jnp.concatenate(
        [h, jnp.ones((1, _PB), jnp.float32),
         jnp.zeros((_W - _F - 1, _PB), jnp.float32)], axis=0)


def _build_prep(interpret=False):
    return pl.pallas_call(
        _prep_body,
        grid=(_NPAD // _PB,),
        in_specs=[pl.BlockSpec((1, _PB), lambda i: (0, i)),
                  pl.BlockSpec((_F, 1), lambda i: (0, 0)),
                  pl.BlockSpec((_F, 1), lambda i: (0, 0))],
        out_specs=pl.BlockSpec((_W, _PB), lambda i: (0, i)),
        out_shape=jax.ShapeDtypeStruct((_W, _NPAD), jnp.float32),
        interpret=interpret,
    )


_mid = _build_mid()
_last = _build_last()
_prep = _build_prep()


def kernel(x, edge_index, batch, in_w, in_b, c1_wl, c1_bl, c1_wr, c2_wl,
           c2_bl, c2_wr, c3_wl, c3_bl, c3_wr, gn2_w, gn2_b, gn2_ms, gn3_w,
           gn3_b, gn3_ms, s1_wih, s1_whh, s1_bih, s1_bhh, s2_wih, s2_whh,
           s2_bih, s2_bhh, s3_wih, s3_whh, s3_bih, s3_bhh, out_w, out_b):
    f32 = jnp.float32
    _sc_agg = _build_sc_agg()
    src2d = edge_index[0].reshape(_NROWS, _EW)
    dst2d = edge_index[1].reshape(_NROWS, _EW)
    batcht = jnp.pad(batch.reshape(1, _N), ((0, 0), (0, _NPAD - _N)),
                     constant_values=_B)
    zrows = jnp.zeros((_TS, _W), f32)
    xt = jnp.pad(x.reshape(1, _N), ((0, 0), (0, _NPAD - _N)))

    def col(v):
        return v.reshape(-1, 1)

    xpadt = _prep(xt, col(in_w.reshape(_F)), col(in_b))

    out = None
    q1 = q2 = None
    layer = [
        (c1_wl, c1_bl, c1_wr, s1_wih, s1_whh, s1_bih, s1_bhh,
         gn2_w, gn2_b, gn2_ms),
        (c2_wl, c2_bl, c2_wr, s2_wih, s2_whh, s2_bih, s2_bhh,
         gn3_w, gn3_b, gn3_ms),
        (c3_wl, c3_bl, c3_wr, s3_wih, s3_whh, s3_bih, s3_bhh,
         None, None, None),
    ]
    for li, (wl, bl, wr, wih, whh, bih, bhh, gw, gb, gms) in enumerate(layer):
        xpad_nm = xpadt.T
        part = _sc_agg(xpad_nm, src2d, dst2d, zrows)
        partt = jnp.transpose(part, (0, 2, 1))
        if li < 2:
            qs, xpadt = _mid(partt, xpadt, batcht, wl.T, col(bl), wr.T,
                             wih, whh, col(bih), col(bhh),
                             col(gw), col(gb), col(gms))
            if li == 0:
                q1 = qs
            else:
                q2 = qs
        else:
            out = _last(partt, xpadt, batcht, wl.T, col(bl), wr.T,
                        wih, whh, col(bih), col(bhh),
                        q1, q2, col(out_w.reshape(-1)), out_b.reshape(1, 1))
    return out.reshape(_B, 1)
